# Initial kernel scaffold; baseline (speedup 1.0000x reference)
import jax
import jax.numpy as jnp
from jax.experimental import pallas as pl


def kernel(learned_energy, perm_1d, W_s, b_s, W_d, b_d, log_temp):
    raise NotImplementedError("write your pallas kernel here")


# speedup vs baseline: 2.1076x; 2.1076x over previous
import functools

import jax
import jax.numpy as jnp
from jax import lax
from jax.experimental import pallas as pl
from jax.experimental.pallas import tpu as pltpu
from jax.experimental.pallas import tpu_sc as plsc


_SC_PARAMS = pltpu.CompilerParams(needs_layout_passes=False)


def _sc_gather(energy, perm):
    b_dim, n_dim = energy.shape
    info = plsc.get_sparse_core_info()
    nc, ns, lanes = info.num_cores, info.num_subcores, info.num_lanes
    nw = nc * ns
    ch = (b_dim * n_dim) // nw
    cpr = n_dim // ch
    mesh = plsc.VectorSubcoreMesh(core_axis_name="c", subcore_axis_name="s")

    @functools.partial(
        pl.kernel,
        mesh=mesh,
        out_type=jax.ShapeDtypeStruct((b_dim * n_dim,), jnp.float32),
        scratch_types=[
            pltpu.VMEM((n_dim,), jnp.float32),
            pltpu.VMEM((ch,), jnp.int32),
            pltpu.VMEM((ch,), jnp.float32),
        ],
        compiler_params=_SC_PARAMS,
    )
    def gk(energy_hbm, perm_hbm, out_hbm, row_v, idx_v, out_v):
        wid = lax.axis_index("s") * nc + lax.axis_index("c")
        b = wid // cpr
        j = wid % cpr
        pltpu.sync_copy(energy_hbm.at[pl.ds(b * n_dim, n_dim)], row_v)
        pltpu.sync_copy(perm_hbm.at[pl.ds(j * ch, ch)], idx_v)

        def body(i, carry):
            idx = idx_v[pl.ds(i * lanes, lanes)]
            out_v[pl.ds(i * lanes, lanes)] = plsc.load_gather(row_v, [idx])
            return carry

        lax.fori_loop(0, ch // lanes, body, 0)
        pltpu.sync_copy(out_v, out_hbm.at[pl.ds(b * n_dim + j * ch, ch)])

    return gk(energy.reshape(-1), perm).reshape(b_dim, n_dim)


def _sc_scatter(pp, perm):
    b_dim, n_dim = pp.shape
    info = plsc.get_sparse_core_info()
    nc, ns, lanes = info.num_cores, info.num_subcores, info.num_lanes
    mesh = plsc.VectorSubcoreMesh(core_axis_name="c", subcore_axis_name="s")

    @functools.partial(
        pl.kernel,
        mesh=mesh,
        out_type=jax.ShapeDtypeStruct((b_dim * n_dim,), jnp.float32),
        scratch_types=[
            pltpu.VMEM((n_dim,), jnp.int32),
            pltpu.VMEM((n_dim,), jnp.float32),
            pltpu.VMEM((n_dim,), jnp.float32),
        ],
        compiler_params=_SC_PARAMS,
    )
    def sk(pp_hbm, perm_hbm, out_hbm, idx_v, val_v, out_v):
        wid = lax.axis_index("s") * nc + lax.axis_index("c")

        @pl.when(wid < b_dim)
        def _():
            pltpu.sync_copy(perm_hbm, idx_v)
            pltpu.sync_copy(pp_hbm.at[pl.ds(wid * n_dim, n_dim)], val_v)

            def body(i, carry):
                idx = idx_v[pl.ds(i * lanes, lanes)]
                vals = val_v[pl.ds(i * lanes, lanes)]
                plsc.store_scatter(out_v, [idx], vals)
                return carry

            lax.fori_loop(0, n_dim // lanes, body, 0)
            pltpu.sync_copy(out_v, out_hbm.at[pl.ds(wid * n_dim, n_dim)])

    return sk(pp.reshape(-1), perm).reshape(b_dim, n_dim)


def _tc_surrogate(eperm, w_s, b_s):
    b_dim, n_dim = eperm.shape
    g_dim = w_s.shape[0]
    c_dim = n_dim // g_dim
    eg = eperm.reshape(b_dim, g_dim, c_dim)
    ev = eperm.reshape(b_dim, 1, n_dim)
    bs2 = b_s.reshape(1, n_dim)

    def ka(eg_ref, ev_ref, ws_ref, bs_ref, surr_ref, dtok_ref, wts_ref,
           tidx_ref, vb_ref):
        egc = eg_ref[0]
        v = ev_ref[0]
        s = lax.dot_general(egc, ws_ref[...], (((0,), (0,)), ((), ())),
                            preferred_element_type=jnp.float32)
        s = s + bs_ref[...]
        surr_ref[0] = s
        m = jnp.max(s, axis=1, keepdims=True)
        e = jnp.exp(s - m)
        l = jnp.sum(e, axis=1, keepdims=True)
        sval = jnp.sum(e * v, axis=1, keepdims=True)
        pos = lax.broadcasted_iota(jnp.int32, (1, n_dim), 1).astype(
            jnp.float32) * (1.0 / n_dim)
        spos = jnp.sum(e * pos, axis=1, keepdims=True)
        one = jnp.ones_like(l)
        dtok_ref[0] = jnp.concatenate([sval / l, spos / l, one / l], axis=1)

        absx = jnp.abs(egc)
        w = jnp.max(absx, axis=0, keepdims=True)
        gi = lax.broadcasted_iota(jnp.int32, (g_dim, c_dim), 0).astype(
            jnp.float32)
        argf = jnp.min(jnp.where(absx == w, gi, float(g_dim)), axis=0,
                       keepdims=True)
        arg = argf.astype(jnp.int32)
        ci = lax.broadcasted_iota(jnp.int32, (1, c_dim), 1)
        tid = arg * c_dim + ci
        valid = (tid >= 0) & (tid < n_dim)
        tidx_ref[0] = jnp.clip(tid, 0, n_dim - 1)
        vb_ref[0] = valid.astype(jnp.int32)
        wts_ref[0] = w * valid.astype(jnp.float32)

    return pl.pallas_call(
        ka,
        grid=(b_dim,),
        in_specs=[
            pl.BlockSpec((1, g_dim, c_dim), lambda b: (b, 0, 0)),
            pl.BlockSpec((1, 1, n_dim), lambda b: (b, 0, 0)),
            pl.BlockSpec((g_dim, n_dim), lambda b: (0, 0)),
            pl.BlockSpec((1, n_dim), lambda b: (0, 0)),
        ],
        out_specs=[
            pl.BlockSpec((1, c_dim, n_dim), lambda b: (b, 0, 0)),
            pl.BlockSpec((1, c_dim, 3), lambda b: (b, 0, 0)),
            pl.BlockSpec((1, 1, c_dim), lambda b: (b, 0, 0)),
            pl.BlockSpec((1, 1, c_dim), lambda b: (b, 0, 0)),
            pl.BlockSpec((1, 1, c_dim), lambda b: (b, 0, 0)),
        ],
        out_shape=[
            jax.ShapeDtypeStruct((b_dim, c_dim, n_dim), jnp.float32),
            jax.ShapeDtypeStruct((b_dim, c_dim, 3), jnp.float32),
            jax.ShapeDtypeStruct((b_dim, 1, c_dim), jnp.float32),
            jax.ShapeDtypeStruct((b_dim, 1, c_dim), jnp.int32),
            jax.ShapeDtypeStruct((b_dim, 1, c_dim), jnp.int32),
        ],
    )(eg, ev, w_s, bs2)


def _tc_decoder(dtok, w_d, b_d, log_temp):
    b_dim, c_dim, _ = dtok.shape
    n_dim = w_d.shape[1]
    bd2 = b_d.reshape(1, n_dim)
    lt2 = log_temp.reshape(1, 1)

    def kb(dtok_ref, wd_ref, bd_ref, lt_ref, dec_ref, scat_ref, pp_ref,
           doubt_ref, supp_ref, st_ref):
        dt = dtok_ref[0]
        wd = wd_ref[...]
        stm = jnp.exp(lt_ref[...])
        st = stm[0, 0]
        d = (dt[:, 0:1] * wd[0:1, :] + dt[:, 1:2] * wd[1:2, :]
             + dt[:, 2:3] * wd[2:3, :]) + bd_ref[...]
        dec_ref[0] = d
        dts = d / st
        m2 = jnp.max(dts, axis=1, keepdims=True)
        e2 = jnp.exp(dts - m2)
        l2 = jnp.sum(e2, axis=1, keepdims=True)
        p = e2 / l2
        scat_ref[0] = p
        h = -jnp.sum(p * jnp.log(p + 1e-9), axis=1, keepdims=True)
        mx = jnp.max(p, axis=1, keepdims=True)
        sm = jnp.sum(p, axis=1, keepdims=True)
        doubt_ref[0] = jnp.concatenate([h, mx, sm], axis=1)
        supp_ref[0] = jnp.sum(p * p, axis=1, keepdims=True)
        vals = dt[:, 0:1]
        pp_ref[0] = jnp.sum(p * vals, axis=0, keepdims=True)
        st_ref[...] = stm

    return pl.pallas_call(
        kb,
        grid=(b_dim,),
        in_specs=[
            pl.BlockSpec((1, c_dim, 3), lambda b: (b, 0, 0)),
            pl.BlockSpec((3, n_dim), lambda b: (0, 0)),
            pl.BlockSpec((1, n_dim), lambda b: (0, 0)),
            pl.BlockSpec((1, 1), lambda b: (0, 0)),
        ],
        out_specs=[
            pl.BlockSpec((1, c_dim, n_dim), lambda b: (b, 0, 0)),
            pl.BlockSpec((1, c_dim, n_dim), lambda b: (b, 0, 0)),
            pl.BlockSpec((1, 1, n_dim), lambda b: (b, 0, 0)),
            pl.BlockSpec((1, c_dim, 3), lambda b: (b, 0, 0)),
            pl.BlockSpec((1, c_dim, 1), lambda b: (b, 0, 0)),
            pl.BlockSpec((1, 1), lambda b: (0, 0)),
        ],
        out_shape=[
            jax.ShapeDtypeStruct((b_dim, c_dim, n_dim), jnp.float32),
            jax.ShapeDtypeStruct((b_dim, c_dim, n_dim), jnp.float32),
            jax.ShapeDtypeStruct((b_dim, 1, n_dim), jnp.float32),
            jax.ShapeDtypeStruct((b_dim, c_dim, 3), jnp.float32),
            jax.ShapeDtypeStruct((b_dim, c_dim, 1), jnp.float32),
            jax.ShapeDtypeStruct((1, 1), jnp.float32),
        ],
    )(dtok, w_d, bd2, lt2)


def kernel(learned_energy, perm_1d, W_s, b_s, W_d, b_d, log_temp):
    b_dim = learned_energy.shape[0]
    n_dim = learned_energy.shape[2]
    c_dim = n_dim // W_s.shape[0]

    energy = learned_energy.reshape(b_dim, n_dim)
    eperm = _sc_gather(energy, perm_1d)
    surr, dtok, wts, tidx, vb = _tc_surrogate(eperm, W_s, b_s)
    dec, scat, pp, doubt, supp, st = _tc_decoder(dtok, W_d, b_d, log_temp)
    proj = _sc_scatter(pp.reshape(b_dim, n_dim), perm_1d)

    return (
        eperm,
        surr,
        dtok,
        dec,
        proj[:, None, :],
        scat,
        doubt,
        supp.reshape(b_dim, c_dim),
        st.reshape(()),
        tidx.reshape(b_dim, c_dim),
        vb.reshape(b_dim, c_dim).astype(bool),
        wts.reshape(b_dim, c_dim),
    )


# speedup vs baseline: 2.2159x; 1.0514x over previous
import functools

import jax
import jax.numpy as jnp
from jax import lax
from jax.experimental import pallas as pl
from jax.experimental.pallas import tpu as pltpu
from jax.experimental.pallas import tpu_sc as plsc


_SC_PARAMS = pltpu.CompilerParams(needs_layout_passes=False)


def _sc_gather(energy, perm):
    b_dim, n_dim = energy.shape
    info = plsc.get_sparse_core_info()
    nc, ns, lanes = info.num_cores, info.num_subcores, info.num_lanes
    nw = nc * ns
    ch = (b_dim * n_dim) // nw
    cpr = n_dim // ch
    mesh = plsc.VectorSubcoreMesh(core_axis_name="c", subcore_axis_name="s")

    @functools.partial(
        pl.kernel,
        mesh=mesh,
        out_type=jax.ShapeDtypeStruct((b_dim * n_dim,), jnp.float32),
        scratch_types=[
            pltpu.VMEM((n_dim,), jnp.float32),
            pltpu.VMEM((ch,), jnp.int32),
            pltpu.VMEM((ch,), jnp.float32),
        ],
        compiler_params=_SC_PARAMS,
    )
    def gk(energy_hbm, perm_hbm, out_hbm, row_v, idx_v, out_v):
        wid = lax.axis_index("s") * nc + lax.axis_index("c")
        b = wid // cpr
        j = wid % cpr
        pltpu.sync_copy(energy_hbm.at[pl.ds(b * n_dim, n_dim)], row_v)
        pltpu.sync_copy(perm_hbm.at[pl.ds(j * ch, ch)], idx_v)

        def body(i, carry):
            idx = idx_v[pl.ds(i * lanes, lanes)]
            out_v[pl.ds(i * lanes, lanes)] = plsc.load_gather(row_v, [idx])
            return carry

        lax.fori_loop(0, ch // lanes, body, 0)
        pltpu.sync_copy(out_v, out_hbm.at[pl.ds(b * n_dim + j * ch, ch)])

    return gk(energy.reshape(-1), perm).reshape(b_dim, n_dim)


def _sc_scatter(pp, perm):
    b_dim, n_dim = pp.shape
    info = plsc.get_sparse_core_info()
    nc, ns, lanes = info.num_cores, info.num_subcores, info.num_lanes
    mesh = plsc.VectorSubcoreMesh(core_axis_name="c", subcore_axis_name="s")

    @functools.partial(
        pl.kernel,
        mesh=mesh,
        out_type=jax.ShapeDtypeStruct((b_dim * n_dim,), jnp.float32),
        scratch_types=[
            pltpu.VMEM((n_dim,), jnp.int32),
            pltpu.VMEM((n_dim,), jnp.float32),
            pltpu.VMEM((n_dim,), jnp.float32),
        ],
        compiler_params=_SC_PARAMS,
    )
    def sk(pp_hbm, perm_hbm, out_hbm, idx_v, val_v, out_v):
        wid = lax.axis_index("s") * nc + lax.axis_index("c")

        @pl.when(wid < b_dim)
        def _():
            pltpu.sync_copy(perm_hbm, idx_v)
            pltpu.sync_copy(pp_hbm.at[pl.ds(wid * n_dim, n_dim)], val_v)

            def body(i, carry):
                idx = idx_v[pl.ds(i * lanes, lanes)]
                vals = val_v[pl.ds(i * lanes, lanes)]
                plsc.store_scatter(out_v, [idx], vals)
                return carry

            lax.fori_loop(0, n_dim // lanes, body, 0)
            pltpu.sync_copy(out_v, out_hbm.at[pl.ds(wid * n_dim, n_dim)])

    return sk(pp.reshape(-1), perm).reshape(b_dim, n_dim)


def _tc_surrogate(eperm, w_s, b_s):
    b_dim, n_dim = eperm.shape
    g_dim = w_s.shape[0]
    c_dim = n_dim // g_dim
    eg = eperm.reshape(b_dim, g_dim, c_dim)
    ev = eperm.reshape(b_dim, 1, n_dim)
    bs2 = b_s.reshape(1, n_dim)

    def ka(eg_ref, ev_ref, ws_ref, bs_ref, pos_ref, surr_ref, dtok_ref,
           wts_ref, tidx_ref, vb_ref):
        egc = eg_ref[0]
        v = ev_ref[0]
        s = lax.dot_general(egc, ws_ref[...], (((0,), (0,)), ((), ())),
                            preferred_element_type=jnp.float32)
        s = s + bs_ref[...]
        surr_ref[0] = s
        m = jnp.max(s, axis=1, keepdims=True)
        e = jnp.exp(s - m)
        l = jnp.sum(e, axis=1, keepdims=True)
        sval = jnp.sum(e * v, axis=1, keepdims=True)
        spos = jnp.sum(e * pos_ref[...], axis=1, keepdims=True)
        one = jnp.ones_like(l)
        dtok_ref[0] = jnp.concatenate([sval / l, spos / l, one / l], axis=1)

        absx = jnp.abs(egc)
        w = jnp.max(absx, axis=0, keepdims=True)
        gi = lax.broadcasted_iota(jnp.int32, (g_dim, c_dim), 0).astype(
            jnp.float32)
        argf = jnp.min(jnp.where(absx == w, gi, float(g_dim)), axis=0,
                       keepdims=True)
        arg = argf.astype(jnp.int32)
        ci = lax.broadcasted_iota(jnp.int32, (1, c_dim), 1)
        tid = arg * c_dim + ci
        valid = (tid >= 0) & (tid < n_dim)
        tidx_ref[0] = jnp.clip(tid, 0, n_dim - 1)
        vb_ref[0] = valid.astype(jnp.int32)
        wts_ref[0] = w * valid.astype(jnp.float32)

    return pl.pallas_call(
        ka,
        grid=(b_dim,),
        in_specs=[
            pl.BlockSpec((1, g_dim, c_dim), lambda b: (b, 0, 0)),
            pl.BlockSpec((1, 1, n_dim), lambda b: (b, 0, 0)),
            pl.BlockSpec((g_dim, n_dim), lambda b: (0, 0)),
            pl.BlockSpec((1, n_dim), lambda b: (0, 0)),
            pl.BlockSpec((1, n_dim), lambda b: (0, 0)),
        ],
        out_specs=[
            pl.BlockSpec((1, c_dim, n_dim), lambda b: (b, 0, 0)),
            pl.BlockSpec((1, c_dim, 3), lambda b: (b, 0, 0)),
            pl.BlockSpec((1, 1, c_dim), lambda b: (b, 0, 0)),
            pl.BlockSpec((1, 1, c_dim), lambda b: (b, 0, 0)),
            pl.BlockSpec((1, 1, c_dim), lambda b: (b, 0, 0)),
        ],
        out_shape=[
            jax.ShapeDtypeStruct((b_dim, c_dim, n_dim), jnp.float32),
            jax.ShapeDtypeStruct((b_dim, c_dim, 3), jnp.float32),
            jax.ShapeDtypeStruct((b_dim, 1, c_dim), jnp.float32),
            jax.ShapeDtypeStruct((b_dim, 1, c_dim), jnp.int32),
            jax.ShapeDtypeStruct((b_dim, 1, c_dim), jnp.int32),
        ],
    )(eg, ev, w_s, bs2,
      (jnp.arange(n_dim, dtype=jnp.float32) / n_dim).reshape(1, n_dim))


def _tc_decoder(dtok, w_d, b_d, log_temp):
    b_dim, c_dim, _ = dtok.shape
    n_dim = w_d.shape[1]
    bd2 = b_d.reshape(1, n_dim)
    lt2 = log_temp.reshape(1, 1)

    def kb(dtok_ref, wd_ref, bd_ref, lt_ref, dec_ref, scat_ref, pp_ref,
           doubt_ref, supp_ref, st_ref):
        dt = dtok_ref[0]
        wd = wd_ref[...]
        stm = jnp.exp(lt_ref[...])
        st = stm[0, 0]
        d = (dt[:, 0:1] * wd[0:1, :] + dt[:, 1:2] * wd[1:2, :]
             + dt[:, 2:3] * wd[2:3, :]) + bd_ref[...]
        dec_ref[0] = d
        dts = d / st
        m2 = jnp.max(dts, axis=1, keepdims=True)
        x = dts - m2
        e2 = jnp.exp(x)
        l2 = jnp.sum(e2, axis=1, keepdims=True)
        scat_ref[0] = e2 / l2
        sxe = jnp.sum(e2 * x, axis=1, keepdims=True)
        se2 = jnp.sum(e2 * e2, axis=1, keepdims=True)
        h = jnp.log(l2) - sxe / l2
        mx = jnp.ones_like(l2) / l2
        sm = l2 / l2
        doubt_ref[0] = jnp.concatenate([h, mx, sm], axis=1)
        supp_ref[0] = se2 / (l2 * l2)
        vals2 = dt[:, 0:1] / l2
        pp_ref[0] = jnp.sum(e2 * vals2, axis=0, keepdims=True)
        st_ref[...] = stm

    return pl.pallas_call(
        kb,
        grid=(b_dim,),
        in_specs=[
            pl.BlockSpec((1, c_dim, 3), lambda b: (b, 0, 0)),
            pl.BlockSpec((3, n_dim), lambda b: (0, 0)),
            pl.BlockSpec((1, n_dim), lambda b: (0, 0)),
            pl.BlockSpec((1, 1), lambda b: (0, 0)),
        ],
        out_specs=[
            pl.BlockSpec((1, c_dim, n_dim), lambda b: (b, 0, 0)),
            pl.BlockSpec((1, c_dim, n_dim), lambda b: (b, 0, 0)),
            pl.BlockSpec((1, 1, n_dim), lambda b: (b, 0, 0)),
            pl.BlockSpec((1, c_dim, 3), lambda b: (b, 0, 0)),
            pl.BlockSpec((1, c_dim, 1), lambda b: (b, 0, 0)),
            pl.BlockSpec((1, 1), lambda b: (0, 0)),
        ],
        out_shape=[
            jax.ShapeDtypeStruct((b_dim, c_dim, n_dim), jnp.float32),
            jax.ShapeDtypeStruct((b_dim, c_dim, n_dim), jnp.float32),
            jax.ShapeDtypeStruct((b_dim, 1, n_dim), jnp.float32),
            jax.ShapeDtypeStruct((b_dim, c_dim, 3), jnp.float32),
            jax.ShapeDtypeStruct((b_dim, c_dim, 1), jnp.float32),
            jax.ShapeDtypeStruct((1, 1), jnp.float32),
        ],
    )(dtok, w_d, bd2, lt2)


def kernel(learned_energy, perm_1d, W_s, b_s, W_d, b_d, log_temp):
    b_dim = learned_energy.shape[0]
    n_dim = learned_energy.shape[2]
    c_dim = n_dim // W_s.shape[0]

    energy = learned_energy.reshape(b_dim, n_dim)
    eperm = _sc_gather(energy, perm_1d)
    surr, dtok, wts, tidx, vb = _tc_surrogate(eperm, W_s, b_s)
    dec, scat, pp, doubt, supp, st = _tc_decoder(dtok, W_d, b_d, log_temp)
    proj = _sc_scatter(pp.reshape(b_dim, n_dim), perm_1d)

    return (
        eperm,
        surr,
        dtok,
        dec,
        proj[:, None, :],
        scat,
        doubt,
        supp.reshape(b_dim, c_dim),
        st.reshape(()),
        tidx.reshape(b_dim, c_dim),
        vb.reshape(b_dim, c_dim).astype(bool),
        wts.reshape(b_dim, c_dim),
    )


# speedup vs baseline: 2.3044x; 1.0399x over previous
import functools

import jax
import jax.numpy as jnp
from jax import lax
from jax.experimental import pallas as pl
from jax.experimental.pallas import tpu as pltpu
from jax.experimental.pallas import tpu_sc as plsc


_SC_PARAMS = pltpu.CompilerParams(needs_layout_passes=False)


def _sc_gather(energy, perm):
    b_dim, n_dim = energy.shape
    info = plsc.get_sparse_core_info()
    nc, ns, lanes = info.num_cores, info.num_subcores, info.num_lanes
    nw = nc * ns
    ch = (b_dim * n_dim) // nw
    cpr = n_dim // ch
    mesh = plsc.VectorSubcoreMesh(core_axis_name="c", subcore_axis_name="s")

    @functools.partial(
        pl.kernel,
        mesh=mesh,
        out_type=jax.ShapeDtypeStruct((b_dim, n_dim), jnp.float32),
        scratch_types=[
            pltpu.VMEM((n_dim,), jnp.float32),
            pltpu.VMEM((ch,), jnp.int32),
            pltpu.VMEM((ch,), jnp.float32),
        ],
        compiler_params=_SC_PARAMS,
    )
    def gk(energy_hbm, perm_hbm, out_hbm, row_v, idx_v, out_v):
        wid = lax.axis_index("s") * nc + lax.axis_index("c")
        b = wid // cpr
        j = wid % cpr
        pltpu.sync_copy(energy_hbm.at[b], row_v)
        pltpu.sync_copy(perm_hbm.at[pl.ds(j * ch, ch)], idx_v)

        def body(i, carry):
            idx = idx_v[pl.ds(i * lanes, lanes)]
            out_v[pl.ds(i * lanes, lanes)] = plsc.load_gather(row_v, [idx])
            return carry

        lax.fori_loop(0, ch // lanes, body, 0)
        pltpu.sync_copy(out_v, out_hbm.at[b, pl.ds(j * ch, ch)])

    return gk(energy, perm)


def _sc_scatter(pp, perm):
    b_dim, n_dim = pp.shape
    info = plsc.get_sparse_core_info()
    nc, ns, lanes = info.num_cores, info.num_subcores, info.num_lanes
    mesh = plsc.VectorSubcoreMesh(core_axis_name="c", subcore_axis_name="s")

    @functools.partial(
        pl.kernel,
        mesh=mesh,
        out_type=jax.ShapeDtypeStruct((b_dim, n_dim), jnp.float32),
        scratch_types=[
            pltpu.VMEM((n_dim,), jnp.int32),
            pltpu.VMEM((n_dim,), jnp.float32),
            pltpu.VMEM((n_dim,), jnp.float32),
        ],
        compiler_params=_SC_PARAMS,
    )
    def sk(pp_hbm, perm_hbm, out_hbm, idx_v, val_v, out_v):
        wid = lax.axis_index("s") * nc + lax.axis_index("c")

        @pl.when(wid < b_dim)
        def _():
            pltpu.sync_copy(perm_hbm, idx_v)
            pltpu.sync_copy(pp_hbm.at[wid], val_v)

            def body(i, carry):
                idx = idx_v[pl.ds(i * lanes, lanes)]
                vals = val_v[pl.ds(i * lanes, lanes)]
                plsc.store_scatter(out_v, [idx], vals)
                return carry

            lax.fori_loop(0, n_dim // lanes, body, 0)
            pltpu.sync_copy(out_v, out_hbm.at[wid])

    return sk(pp, perm)


def _tc_surrogate(eperm, w_s, b_s):
    b_dim, n_dim = eperm.shape
    g_dim = w_s.shape[0]
    c_dim = n_dim // g_dim
    bs2 = b_s.reshape(1, n_dim)

    def ka(ev_ref, ws_ref, bs_ref, pos_ref, surr_ref, dtok_ref,
           wts_ref, tidx_ref, vb_ref):
        b = pl.program_id(0)
        v = ev_ref[pl.ds(b, 1), :]
        egc = v.reshape(g_dim, c_dim)
        s = lax.dot_general(egc, ws_ref[...], (((0,), (0,)), ((), ())),
                            preferred_element_type=jnp.float32)
        s = s + bs_ref[...]
        surr_ref[0] = s
        m = jnp.max(s, axis=1, keepdims=True)
        e = jnp.exp(s - m)
        l = jnp.sum(e, axis=1, keepdims=True)
        sval = jnp.sum(e * v, axis=1, keepdims=True)
        spos = jnp.sum(e * pos_ref[...], axis=1, keepdims=True)
        one = jnp.ones_like(l)
        dtok_ref[0] = jnp.concatenate([sval / l, spos / l, one / l], axis=1)

        absx = jnp.abs(egc)
        w = jnp.max(absx, axis=0, keepdims=True)
        gi = lax.broadcasted_iota(jnp.int32, (g_dim, c_dim), 0).astype(
            jnp.float32)
        argf = jnp.min(jnp.where(absx == w, gi, float(g_dim)), axis=0,
                       keepdims=True)
        arg = argf.astype(jnp.int32)
        ci = lax.broadcasted_iota(jnp.int32, (1, c_dim), 1)
        tid = arg * c_dim + ci
        valid = (tid >= 0) & (tid < n_dim)
        tidx_ref[0] = jnp.clip(tid, 0, n_dim - 1)
        vb_ref[0] = valid.astype(jnp.int32)
        wts_ref[0] = w * valid.astype(jnp.float32)

    return pl.pallas_call(
        ka,
        grid=(b_dim,),
        in_specs=[
            pl.BlockSpec((b_dim, n_dim), lambda b: (0, 0)),
            pl.BlockSpec((g_dim, n_dim), lambda b: (0, 0)),
            pl.BlockSpec((1, n_dim), lambda b: (0, 0)),
            pl.BlockSpec((1, n_dim), lambda b: (0, 0)),
        ],
        out_specs=[
            pl.BlockSpec((1, c_dim, n_dim), lambda b: (b, 0, 0)),
            pl.BlockSpec((1, c_dim, 3), lambda b: (b, 0, 0)),
            pl.BlockSpec((1, 1, c_dim), lambda b: (b, 0, 0)),
            pl.BlockSpec((1, 1, c_dim), lambda b: (b, 0, 0)),
            pl.BlockSpec((1, 1, c_dim), lambda b: (b, 0, 0)),
        ],
        out_shape=[
            jax.ShapeDtypeStruct((b_dim, c_dim, n_dim), jnp.float32),
            jax.ShapeDtypeStruct((b_dim, c_dim, 3), jnp.float32),
            jax.ShapeDtypeStruct((b_dim, 1, c_dim), jnp.float32),
            jax.ShapeDtypeStruct((b_dim, 1, c_dim), jnp.int32),
            jax.ShapeDtypeStruct((b_dim, 1, c_dim), jnp.int32),
        ],
    )(eperm, w_s, bs2,
      (jnp.arange(n_dim, dtype=jnp.float32) / n_dim).reshape(1, n_dim))


def _tc_decoder(dtok, w_d, b_d, log_temp):
    b_dim, c_dim, _ = dtok.shape
    n_dim = w_d.shape[1]
    bd2 = b_d.reshape(1, n_dim)
    lt2 = log_temp.reshape(1, 1)

    def kb(dtok_ref, wd_ref, bd_ref, lt_ref, dec_ref, scat_ref, pp_ref,
           doubt_ref, supp_ref, st_ref):
        dt = dtok_ref[0]
        wd = wd_ref[...]
        stm = jnp.exp(lt_ref[...])
        st = stm[0, 0]
        d = (dt[:, 0:1] * wd[0:1, :] + dt[:, 1:2] * wd[1:2, :]
             + dt[:, 2:3] * wd[2:3, :]) + bd_ref[...]
        dec_ref[0] = d
        dts = d / st
        m2 = jnp.max(dts, axis=1, keepdims=True)
        x = dts - m2
        e2 = jnp.exp(x)
        l2 = jnp.sum(e2, axis=1, keepdims=True)
        scat_ref[0] = e2 / l2
        sxe = jnp.sum(e2 * x, axis=1, keepdims=True)
        se2 = jnp.sum(e2 * e2, axis=1, keepdims=True)
        h = jnp.log(l2) - sxe / l2
        mx = jnp.ones_like(l2) / l2
        sm = l2 / l2
        doubt_ref[0] = jnp.concatenate([h, mx, sm], axis=1)
        supp_ref[0] = se2 / (l2 * l2)
        vals2 = dt[:, 0:1] / l2
        b = pl.program_id(0)
        pp_ref[pl.ds(b, 1), :] = jnp.sum(e2 * vals2, axis=0, keepdims=True)
        st_ref[...] = stm

    return pl.pallas_call(
        kb,
        grid=(b_dim,),
        in_specs=[
            pl.BlockSpec((1, c_dim, 3), lambda b: (b, 0, 0)),
            pl.BlockSpec((3, n_dim), lambda b: (0, 0)),
            pl.BlockSpec((1, n_dim), lambda b: (0, 0)),
            pl.BlockSpec((1, 1), lambda b: (0, 0)),
        ],
        out_specs=[
            pl.BlockSpec((1, c_dim, n_dim), lambda b: (b, 0, 0)),
            pl.BlockSpec((1, c_dim, n_dim), lambda b: (b, 0, 0)),
            pl.BlockSpec((b_dim, n_dim), lambda b: (0, 0)),
            pl.BlockSpec((1, c_dim, 3), lambda b: (b, 0, 0)),
            pl.BlockSpec((1, c_dim, 1), lambda b: (b, 0, 0)),
            pl.BlockSpec((1, 1), lambda b: (0, 0)),
        ],
        out_shape=[
            jax.ShapeDtypeStruct((b_dim, c_dim, n_dim), jnp.float32),
            jax.ShapeDtypeStruct((b_dim, c_dim, n_dim), jnp.float32),
            jax.ShapeDtypeStruct((b_dim, n_dim), jnp.float32),
            jax.ShapeDtypeStruct((b_dim, c_dim, 3), jnp.float32),
            jax.ShapeDtypeStruct((b_dim, c_dim, 1), jnp.float32),
            jax.ShapeDtypeStruct((1, 1), jnp.float32),
        ],
    )(dtok, w_d, bd2, lt2)


def kernel(learned_energy, perm_1d, W_s, b_s, W_d, b_d, log_temp):
    b_dim = learned_energy.shape[0]
    n_dim = learned_energy.shape[2]
    c_dim = n_dim // W_s.shape[0]

    energy = learned_energy.reshape(b_dim, n_dim)
    eperm = _sc_gather(energy, perm_1d)
    surr, dtok, wts, tidx, vb = _tc_surrogate(eperm, W_s, b_s)
    dec, scat, pp, doubt, supp, st = _tc_decoder(dtok, W_d, b_d, log_temp)
    proj = _sc_scatter(pp, perm_1d)

    return (
        eperm,
        surr,
        dtok,
        dec,
        proj[:, None, :],
        scat,
        doubt,
        supp.reshape(b_dim, c_dim),
        st.reshape(()),
        tidx.reshape(b_dim, c_dim),
        vb.reshape(b_dim, c_dim).astype(bool),
        wts.reshape(b_dim, c_dim),
    )
